# baseline (speedup 1.0000x reference)
import functools

import jax
import jax.numpy as jnp
from jax import lax
from jax.experimental import pallas as pl
from jax.experimental.pallas import tpu as pltpu
from jax.experimental.pallas import tpu_sc as plsc

BATCH = 16384
HIDDEN = 1024
N_TAB = 1000
ROWS_PER_TILE = 64
GROUP = 16


@jax.jit
def _embed(labels, table):
    info = plsc.get_sparse_core_info()
    num_workers = info.num_cores * info.num_subcores
    b_per_w = BATCH // num_workers
    n_groups = b_per_w // GROUP

    table_flat = table.reshape(-1)
    mesh = plsc.VectorSubcoreMesh(core_axis_name="c", subcore_axis_name="s")

    @functools.partial(
        pl.kernel,
        mesh=mesh,
        out_type=jax.ShapeDtypeStruct((BATCH * HIDDEN,), jnp.float32),
        scratch_types=[
            pltpu.VMEM((b_per_w,), jnp.int32),
            pltpu.VMEM((HIDDEN,), jnp.float32),
            pltpu.VMEM_SHARED((N_TAB * HIDDEN,), jnp.float32),
            pltpu.SemaphoreType.DMA,
        ],
    )
    def k(labels_hbm, tabf_hbm, outf_hbm, idx_v, dummy_v, tab_sh, sem):
        cid = lax.axis_index("c")
        sid = lax.axis_index("s")
        wid = sid * info.num_cores + cid
        base = wid * b_per_w
        off = jnp.minimum(sid * ROWS_PER_TILE, N_TAB - ROWS_PER_TILE) * HIDDEN
        pltpu.sync_copy(
            tabf_hbm.at[pl.ds(off, ROWS_PER_TILE * HIDDEN)],
            tab_sh.at[pl.ds(off, ROWS_PER_TILE * HIDDEN)],
        )
        pltpu.sync_copy(labels_hbm.at[pl.ds(base, b_per_w)], idx_v)
        plsc.subcore_barrier()

        def fire(g, carry):
            labs = idx_v[pl.ds(g * GROUP, GROUP)]
            for lane in range(GROUP):
                row = labs[lane] * HIDDEN
                dst_off = (base + g * GROUP + lane) * HIDDEN
                pltpu.async_copy(
                    tab_sh.at[pl.ds(row, HIDDEN)],
                    outf_hbm.at[pl.ds(dst_off, HIDDEN)],
                    sem,
                ).wait()
            return carry

        lax.fori_loop(0, n_groups, fire, 0)

    return k(labels, table_flat).reshape(BATCH, HIDDEN)


def kernel(labels, table):
    return _embed(labels, table)


# speedup vs baseline: 3.2734x; 3.2734x over previous
import functools

import jax
import jax.numpy as jnp
from jax import lax
from jax.experimental import pallas as pl
from jax.experimental.pallas import tpu as pltpu
from jax.experimental.pallas import tpu_sc as plsc

BATCH = 16384
HIDDEN = 1024
N_TAB = 1000
ROWS_PER_TILE = 64
GROUP = 16


@jax.jit
def _embed(labels, table):
    info = plsc.get_sparse_core_info()
    num_workers = info.num_cores * info.num_subcores
    b_per_w = BATCH // num_workers
    n_groups = b_per_w // GROUP

    table_flat = table.reshape(-1)
    mesh = plsc.VectorSubcoreMesh(core_axis_name="c", subcore_axis_name="s")

    @functools.partial(
        pl.kernel,
        mesh=mesh,
        out_type=jax.ShapeDtypeStruct((BATCH * HIDDEN,), jnp.float32),
        scratch_types=[
            pltpu.VMEM((b_per_w,), jnp.int32),
            pltpu.VMEM((HIDDEN,), jnp.float32),
            pltpu.VMEM_SHARED((N_TAB * HIDDEN,), jnp.float32),
            pltpu.SemaphoreType.DMA,
        ],
    )
    def k(labels_hbm, tabf_hbm, outf_hbm, idx_v, dummy_v, tab_sh, sem):
        cid = lax.axis_index("c")
        sid = lax.axis_index("s")
        wid = sid * info.num_cores + cid
        base = wid * b_per_w
        off = jnp.minimum(sid * ROWS_PER_TILE, N_TAB - ROWS_PER_TILE) * HIDDEN
        pltpu.sync_copy(
            tabf_hbm.at[pl.ds(off, ROWS_PER_TILE * HIDDEN)],
            tab_sh.at[pl.ds(off, ROWS_PER_TILE * HIDDEN)],
        )
        pltpu.sync_copy(labels_hbm.at[pl.ds(base, b_per_w)], idx_v)
        plsc.subcore_barrier()

        def fire(g):
            labs = idx_v[pl.ds(g * GROUP, GROUP)]
            for lane in range(GROUP):
                row = labs[lane] * HIDDEN
                dst_off = (base + g * GROUP + lane) * HIDDEN
                pltpu.async_copy(
                    tab_sh.at[pl.ds(row, HIDDEN)],
                    outf_hbm.at[pl.ds(dst_off, HIDDEN)],
                    sem,
                )

        drain = pltpu.make_async_copy(
            tab_sh.at[pl.ds(0, HIDDEN)],
            outf_hbm.at[pl.ds(base * HIDDEN, HIDDEN)],
            sem,
        )

        def drain_group():
            for _ in range(GROUP):
                drain.wait()

        fire(0)

        def body(g, carry):
            fire(g + 1)
            drain_group()
            return carry

        lax.fori_loop(0, n_groups - 1, body, 0)
        drain_group()

    return k(labels, table_flat).reshape(BATCH, HIDDEN)


def kernel(labels, table):
    return _embed(labels, table)
